# Initial kernel scaffold; baseline (speedup 1.0000x reference)
import jax
import jax.numpy as jnp
from jax.experimental import pallas as pl


def kernel(pos_idx, pe):
    raise NotImplementedError("write your pallas kernel here")


# speedup vs baseline: 2.3783x; 2.3783x over previous
import functools

import jax
import jax.numpy as jnp
from jax import lax
from jax.experimental import pallas as pl
from jax.experimental.pallas import tpu as pltpu
from jax.experimental.pallas import tpu_sc as plsc

D = 1024
TOT = 4 * 8192
NC, NS = 2, 16
NW = NC * NS
PER_W = TOT // NW
C = 32
NCHUNK = PER_W // C

_mesh = plsc.VectorSubcoreMesh(
    core_axis_name="c", subcore_axis_name="s", num_cores=NC, num_subcores=NS
)


@functools.partial(
    pl.kernel,
    mesh=_mesh,
    out_type=jax.ShapeDtypeStruct((TOT, D), jnp.float32),
    scratch_types=[
        pltpu.VMEM((NCHUNK, C), jnp.int32),
        pltpu.VMEM((C, D), jnp.float32),
        pltpu.VMEM((C, D), jnp.float32),
        pltpu.SemaphoreType.DMA,
        pltpu.SemaphoreType.DMA,
    ],
)
def _gather_rows(idx_hbm, table_hbm, out_hbm, idx_v, buf0, buf1, sem0, sem1):
    cid = lax.axis_index("c")
    sid = lax.axis_index("s")
    wid = sid * NC + cid
    base = wid * PER_W

    pltpu.sync_copy(idx_hbm.at[wid], idx_v)

    bufs = (buf0, buf1)
    sems = (sem0, sem1)

    def gather(chunk, b):
        return pltpu.make_async_copy(
            table_hbm.at[idx_v.at[chunk]], bufs[b], sems[b]
        )

    gather(0, 0).start()
    gather(1, 1).start()

    def body(i, _):
        g = i * 2
        for b in range(2):
            chunk = g + b
            gather(chunk, b).wait()
            pltpu.sync_copy(bufs[b], out_hbm.at[pl.ds(base + chunk * C, C)])

            @pl.when(chunk + 2 < NCHUNK)
            def _():
                gather(chunk + 2, b).start()

        return 0

    lax.fori_loop(0, NCHUNK // 2, body, 0)


def kernel(pos_idx, pe):
    idx = pos_idx.astype(jnp.int32).reshape(NW, NCHUNK, C)
    out = _gather_rows(idx, pe)
    return out.reshape(pos_idx.shape + (D,))
